# baseline (speedup 1.0000x reference)
import functools

import jax
import jax.numpy as jnp
from jax import lax
from jax.experimental import pallas as pl
from jax.experimental.pallas import tpu as pltpu
from jax.experimental.pallas import tpu_sc as plsc

_F32 = jnp.float32


_Y = [[16, 17, 18], [32, 33, 34]]
_Z = [[48 + 16 * (3 * i + j) for j in range(3)] for i in range(2)]

_ASSIGN = (
    [(0, 2), (1, 0), (2, 1)]
    + list(zip(range(16, 22),
             [_Y[0][0], _Y[1][1], _Y[1][0], _Y[0][1], _Y[0][0], _Y[0][1]]))
    + list(zip(range(22, 28),
               [_Y[0][1], _Y[0][0], _Y[0][2], _Y[1][0], _Y[1][2], _Y[1][1]]))
    + list(zip(range(32, 48),
               [_Z[0][0] + k for k in range(4)] * 2
               + [_Z[1][0] + k for k in range(4)] * 2))
    + list(zip(range(48, 60),
               [_Z[i][j] for i in range(2) for j in range(3)
                for _ in range(2)]))
)


@functools.partial(
    pl.kernel,
    out_type=jax.ShapeDtypeStruct((64,), _F32),
    mesh=plsc.ScalarSubcoreMesh(axis_name="c", num_cores=1),
    scratch_types=[
        pltpu.SMEM((144,), _F32),
        pltpu.SMEM((64,), _F32),
    ],
)
def _gather_kernel(win_hbm, out_hbm, winbuf, obuf):
    pltpu.sync_copy(win_hbm, winbuf)
    for dst, src in _ASSIGN:
        obuf[dst] = winbuf[src]
    pltpu.sync_copy(obuf, out_hbm)


def _fmt_body(p_ref, a_ref, b_ref, c_ref, d_ref, e_ref):
    a_ref[...] = p_ref[pl.ds(0, 3)]
    for r in range(3):
        b_ref[r, :] = p_ref[pl.ds(16 + 2 * r, 2)]
    for r in range(2):
        c_ref[r, :] = p_ref[pl.ds(22 + 3 * r, 3)]
    for i in range(2):
        for j in range(2):
            d_ref[i, j, :] = p_ref[pl.ds(32 + 8 * i + 4 * j, 4)]
    for i in range(2):
        for j in range(3):
            e_ref[i, j, :] = p_ref[pl.ds(48 + 6 * i + 2 * j, 2)]


_fmt = pl.pallas_call(
    _fmt_body,
    out_shape=(
        jax.ShapeDtypeStruct((3,), _F32),
        jax.ShapeDtypeStruct((3, 2), _F32),
        jax.ShapeDtypeStruct((2, 3), _F32),
        jax.ShapeDtypeStruct((2, 2, 4), _F32),
        jax.ShapeDtypeStruct((2, 3, 2), _F32),
    ),
)


def kernel(x, y, z):
    i = jnp.arange(144, dtype=jnp.int32)
    xg = x[jnp.minimum(i, 15)]
    yg = y[jnp.clip((i - 16) // 16, 0, 1), (i - 16) % 16]
    r = jnp.clip((i - 48) // 16, 0, 5)
    zg = z[r // 3, r % 3, (i - 48) % 16]
    win = jnp.where(i < 16, xg, jnp.where(i < 48, yg, zg))
    packed = _gather_kernel(win)
    return _fmt(packed)


# speedup vs baseline: 1.5414x; 1.5414x over previous
import functools

import jax
import jax.numpy as jnp
from jax import lax
from jax.experimental import pallas as pl
from jax.experimental.pallas import tpu as pltpu
from jax.experimental.pallas import tpu_sc as plsc

_F32 = jnp.float32


_Y = [[16, 17, 18], [32, 33, 34]]
_Z = [[48 + 16 * (3 * i + j) for j in range(3)] for i in range(2)]

_ASSIGN = (
    [(0, 2), (1, 0), (2, 1)]
    + list(zip(range(16, 22),
             [_Y[0][0], _Y[1][1], _Y[1][0], _Y[0][1], _Y[0][0], _Y[0][1]]))
    + list(zip(range(22, 28),
               [_Y[0][1], _Y[0][0], _Y[0][2], _Y[1][0], _Y[1][2], _Y[1][1]]))
    + list(zip(range(32, 48),
               [_Z[0][0] + k for k in range(4)] * 2
               + [_Z[1][0] + k for k in range(4)] * 2))
    + list(zip(range(48, 60),
               [_Z[i][j] for i in range(2) for j in range(3)
                for _ in range(2)]))
)


@functools.partial(
    pl.kernel,
    out_type=jax.ShapeDtypeStruct((64,), _F32),
    mesh=plsc.ScalarSubcoreMesh(axis_name="c", num_cores=1),
    scratch_types=[
        pltpu.SMEM((144,), _F32),
        pltpu.SMEM((64,), _F32),
    ],
)
def _gather_kernel(win_hbm, out_hbm, winbuf, obuf):
    pltpu.sync_copy(win_hbm, winbuf)
    for dst, src in _ASSIGN:
        obuf[dst] = winbuf[src]
    pltpu.sync_copy(obuf, out_hbm)


def _fmt_body(p_ref, a_ref, b_ref, c_ref, d_ref, e_ref):
    a_ref[...] = p_ref[pl.ds(0, 3)]
    for r in range(3):
        b_ref[r, :] = p_ref[pl.ds(16 + 2 * r, 2)]
    for r in range(2):
        c_ref[r, :] = p_ref[pl.ds(22 + 3 * r, 3)]
    for i in range(2):
        for j in range(2):
            d_ref[i, j, :] = p_ref[pl.ds(32 + 8 * i + 4 * j, 4)]
    for i in range(2):
        for j in range(3):
            e_ref[i, j, :] = p_ref[pl.ds(48 + 6 * i + 2 * j, 2)]


_fmt = pl.pallas_call(
    _fmt_body,
    out_shape=(
        jax.ShapeDtypeStruct((3,), _F32),
        jax.ShapeDtypeStruct((3, 2), _F32),
        jax.ShapeDtypeStruct((2, 3), _F32),
        jax.ShapeDtypeStruct((2, 2, 4), _F32),
        jax.ShapeDtypeStruct((2, 3, 2), _F32),
    ),
)


def kernel(x, y, z):
    win = jnp.concatenate(
        [lax.slice(x, (0,), (16,)),
         lax.slice(y, (0, 0), (2, 16)).reshape(32),
         lax.slice(z, (0, 0, 0), (2, 3, 16)).reshape(96)])
    packed = _gather_kernel(win)
    return _fmt(packed)


# speedup vs baseline: 1.7593x; 1.1413x over previous
import functools

import jax
import jax.numpy as jnp
from jax import lax
from jax.experimental import pallas as pl
from jax.experimental.pallas import tpu as pltpu
from jax.experimental.pallas import tpu_sc as plsc

_F32 = jnp.float32


_Y = [[16, 17, 18], [32, 33, 34]]
_Z = [[48 + 16 * (3 * i + j) for j in range(3)] for i in range(2)]

_ASSIGN = (
    [(0, 2), (1, 0), (2, 1)]
    + list(zip(range(16, 22),
             [_Y[0][0], _Y[1][0], _Y[0][0], _Y[1][1], _Y[0][1], _Y[0][1]]))
    + list(zip(range(22, 28),
               [_Y[0][1], _Y[0][0], _Y[0][2], _Y[1][0], _Y[1][2], _Y[1][1]]))
    + list(zip(range(32, 48),
               [_Z[0][0] + k for k in range(4)] * 2
               + [_Z[1][0] + k for k in range(4)] * 2))
    + list(zip(range(48, 60),
               [_Z[i][j] for i in range(2) for _ in range(2)
                for j in range(3)]))
)


@functools.partial(
    pl.kernel,
    out_type=jax.ShapeDtypeStruct((64,), _F32),
    mesh=plsc.ScalarSubcoreMesh(axis_name="c", num_cores=1),
    scratch_types=[
        pltpu.SMEM((144,), _F32),
        pltpu.SMEM((64,), _F32),
    ],
)
def _gather_kernel(win_hbm, out_hbm, winbuf, obuf):
    pltpu.sync_copy(win_hbm, winbuf)
    for dst, src in _ASSIGN:
        obuf[dst] = winbuf[src]
    pltpu.sync_copy(obuf, out_hbm)


def _fmt_body(p_ref, a_ref, bt_ref, c_ref, d_ref, ep_ref):
    a_ref[...] = p_ref[pl.ds(0, 3)]
    for j in range(2):
        bt_ref[j, :] = p_ref[pl.ds(16 + 3 * j, 3)]
    for r in range(2):
        c_ref[r, :] = p_ref[pl.ds(22 + 3 * r, 3)]
    for i in range(2):
        for j in range(2):
            d_ref[i, j, :] = p_ref[pl.ds(32 + 8 * i + 4 * j, 4)]
    for i in range(2):
        for k in range(2):
            ep_ref[i, k, :] = p_ref[pl.ds(48 + 6 * i + 3 * k, 3)]


_fmt = pl.pallas_call(
    _fmt_body,
    out_shape=(
        jax.ShapeDtypeStruct((3,), _F32),
        jax.ShapeDtypeStruct((2, 3), _F32),
        jax.ShapeDtypeStruct((2, 3), _F32),
        jax.ShapeDtypeStruct((2, 2, 4), _F32),
        jax.ShapeDtypeStruct((2, 2, 3), _F32),
    ),
)


def kernel(x, y, z):
    win = jnp.concatenate(
        [lax.slice(x, (0,), (16,)),
         lax.slice(y, (0, 0), (2, 16)).reshape(32),
         lax.slice(z, (0, 0, 0), (2, 3, 16)).reshape(96)])
    packed = _gather_kernel(win)
    a, bt, c, d, ep = _fmt(packed)
    return (a, bt.T, c, d, jnp.transpose(ep, (0, 2, 1)))
